# baseline (speedup 1.0000x reference)
import functools

import jax
import jax.numpy as jnp
import numpy as np
from jax import lax
from jax.experimental import pallas as pl
from jax.experimental.pallas import tpu as pltpu
from jax.experimental.pallas import tpu_sc as plsc

_MAX_MASK_SIZE = 27
_rng = np.random.RandomState(0)
_MASK = int(_rng.randint(0, _MAX_MASK_SIZE))
_START = int(_rng.randint(0, 128 - _MASK))

_B, _F, _T = 128, 128, 2048
_S = _F * _T
_A_LEN = _START * _T
_Z_OFF = _A_LEN
_Z_LEN = _MASK * _T
_B_OFF = _Z_OFF + _Z_LEN
_B_LEN = _S - _B_OFF

_NC, _NS = 2, 16
_NW = _NC * _NS
_BPW = _B // _NW

_CHUNK = 48128
_BATCH_CHUNKS = [
    (0, _CHUNK),
    (_CHUNK, _A_LEN - _CHUNK),
    (_B_OFF, _CHUNK),
    (_B_OFF + _CHUNK, _CHUNK),
    (_B_OFF + 2 * _CHUNK, _B_LEN - 2 * _CHUNK),
]


def _sc_body(x_hbm, o_hbm, buf0, buf1, zbuf, isem0, isem1, osem0, osem1, zsem):
    wid = lax.axis_index("s") * _NC + lax.axis_index("c")
    base = pl.multiple_of(wid * (_BPW * _S), 512)

    chunks = []
    for b in range(_BPW):
        for off, ln in _BATCH_CHUNKS:
            chunks.append((b * _S + off, ln))
    n = len(chunks)
    bufs = (buf0, buf1)
    isems = (isem0, isem1)
    osems = (osem0, osem1)

    def in_cp(i):
        off, ln = chunks[i]
        return pltpu.make_async_copy(
            x_hbm.at[pl.ds(base + off, ln)],
            bufs[i % 2].at[pl.ds(0, ln)],
            isems[i % 2])

    def out_cp(i):
        off, ln = chunks[i]
        return pltpu.make_async_copy(
            bufs[i % 2].at[pl.ds(0, ln)],
            o_hbm.at[pl.ds(base + off, ln)],
            osems[i % 2])

    in_cp(0).start()
    in_cp(1).start()

    def zr(i, _):
        zbuf[pl.ds(i * 16, 16)] = jnp.zeros((16,), jnp.float32)
        return 0
    lax.fori_loop(0, _Z_LEN // 16, zr, 0)
    zcps = [pltpu.make_async_copy(
        zbuf, o_hbm.at[pl.ds(base + b * _S + _Z_OFF, _Z_LEN)], zsem)
        for b in range(_BPW)]
    for c in zcps:
        c.start()

    for i in range(n):
        in_cp(i).wait()
        out_cp(i).start()
        if i + 2 < n:
            out_cp(i).wait()
            in_cp(i + 2).start()
    out_cp(n - 2).wait()
    out_cp(n - 1).wait()
    for c in zcps:
        c.wait()


@functools.partial(jax.jit, static_argnums=())
def _sc_mask_copy(x_flat):
    k = functools.partial(
        pl.kernel,
        mesh=plsc.VectorSubcoreMesh(core_axis_name="c", subcore_axis_name="s"),
        out_type=jax.ShapeDtypeStruct((_B * _S,), jnp.float32),
        scratch_types=[
            pltpu.VMEM((_CHUNK,), jnp.float32),
            pltpu.VMEM((_CHUNK,), jnp.float32),
            pltpu.VMEM((_Z_LEN,), jnp.float32),
            pltpu.SemaphoreType.DMA,
            pltpu.SemaphoreType.DMA,
            pltpu.SemaphoreType.DMA,
            pltpu.SemaphoreType.DMA,
            pltpu.SemaphoreType.DMA,
        ],
    )(_sc_body)
    return k(x_flat)


def kernel(x):
    out_flat = _sc_mask_copy(x.reshape(-1))
    return out_flat.reshape(x.shape)


# speedup vs baseline: 1.0234x; 1.0234x over previous
import functools

import jax
import jax.numpy as jnp
import numpy as np
from jax import lax
from jax.experimental import pallas as pl
from jax.experimental.pallas import tpu as pltpu
from jax.experimental.pallas import tpu_sc as plsc

_MAX_MASK_SIZE = 27
_rng = np.random.RandomState(0)
_MASK = int(_rng.randint(0, _MAX_MASK_SIZE))
_START = int(_rng.randint(0, 128 - _MASK))

_B, _F, _T = 128, 128, 2048
_S = _F * _T
_A_LEN = _START * _T
_Z_OFF = _A_LEN
_Z_LEN = _MASK * _T
_B_OFF = _Z_OFF + _Z_LEN
_B_LEN = _S - _B_OFF

_NC, _NS = 2, 16
_NW = _NC * _NS
_BPW = _B // _NW

_CHUNK = 61440
_ZBUF = 2048
_BATCH_CHUNKS = [
    (0, _CHUNK),
    (_CHUNK, _A_LEN - _CHUNK),
    (_B_OFF, _CHUNK),
    (_B_OFF + _CHUNK, _CHUNK),
    (_B_OFF + 2 * _CHUNK, _B_LEN - 2 * _CHUNK),
]


def _sc_body(x_hbm, o_hbm, shared, zbuf, isem0, isem1, osem0, osem1, zsem):
    cid = lax.axis_index("c")
    sid = lax.axis_index("s")
    wid = sid * _NC + cid
    base = pl.multiple_of(wid * (_BPW * _S), 512)

    chunks = []
    for b in range(_BPW):
        for off, ln in _BATCH_CHUNKS:
            chunks.append((b * _S + off, ln))
    n = len(chunks)
    isems = (isem0, isem1)
    osems = (osem0, osem1)

    def in_cp(i):
        off, ln = chunks[i]
        return pltpu.make_async_copy(
            x_hbm.at[pl.ds(base + off, ln)],
            shared.at[sid, i % 2, pl.ds(0, ln)],
            isems[i % 2])

    def out_cp(i):
        off, ln = chunks[i]
        return pltpu.make_async_copy(
            shared.at[sid, i % 2, pl.ds(0, ln)],
            o_hbm.at[pl.ds(base + off, ln)],
            osems[i % 2])

    in_cp(0).start()
    in_cp(1).start()

    def zr(i, _):
        zbuf[pl.ds(i * 16, 16)] = jnp.zeros((16,), jnp.float32)
        return 0
    lax.fori_loop(0, _ZBUF // 16, zr, 0)
    zcps = [pltpu.make_async_copy(
        zbuf, o_hbm.at[pl.ds(base + b * _S + _Z_OFF + z * _ZBUF, _ZBUF)], zsem)
        for b in range(_BPW) for z in range(_Z_LEN // _ZBUF)]
    for c in zcps:
        c.start()

    for i in range(n):
        in_cp(i).wait()
        out_cp(i).start()
        if i + 2 < n:
            out_cp(i).wait()
            in_cp(i + 2).start()
    out_cp(n - 2).wait()
    out_cp(n - 1).wait()
    for c in zcps:
        c.wait()


@functools.partial(jax.jit, static_argnums=())
def _sc_mask_copy(x_flat):
    k = functools.partial(
        pl.kernel,
        mesh=plsc.VectorSubcoreMesh(core_axis_name="c", subcore_axis_name="s"),
        out_type=jax.ShapeDtypeStruct((_B * _S,), jnp.float32),
        scratch_types=[
            pltpu.VMEM_SHARED((_NS, 2, _CHUNK), jnp.float32),
            pltpu.VMEM((_ZBUF,), jnp.float32),
            pltpu.SemaphoreType.DMA,
            pltpu.SemaphoreType.DMA,
            pltpu.SemaphoreType.DMA,
            pltpu.SemaphoreType.DMA,
            pltpu.SemaphoreType.DMA,
        ],
    )(_sc_body)
    return k(x_flat)


def kernel(x):
    out_flat = _sc_mask_copy(x.reshape(-1))
    return out_flat.reshape(x.shape)


# speedup vs baseline: 2.9513x; 2.8839x over previous
import functools

import jax
import jax.numpy as jnp
import numpy as np
from jax import lax
from jax.experimental import pallas as pl
from jax.experimental.pallas import tpu as pltpu
from jax.experimental.pallas import tpu_sc as plsc

_MAX_MASK_SIZE = 27
_rng = np.random.RandomState(0)
_MASK = int(_rng.randint(0, _MAX_MASK_SIZE))
_START = int(_rng.randint(0, 128 - _MASK))
_END = _START + _MASK

_B, _F, _T = 128, 128, 2048

_NC, _NS = 2, 16
_NW = _NC * _NS
_BPW = _B // _NW

_CR = 16
_ROW_CHUNKS = [
    (0, 16), (16, 16), (32, 8),
    (64, 16), (80, 16), (96, 16), (112, 16),
]
_NCH = len(_ROW_CHUNKS)
_BND0, _BNDR = 40, 24


def _sc_body(x_hbm, o_hbm, shared, bbuf, isem0, isem1, osem0, osem1, bsem):
    cid = lax.axis_index("c")
    sid = lax.axis_index("s")
    wid = sid * _NC + cid
    b0 = wid * _BPW

    chunks = []
    for b in range(_BPW):
        for r0, nr in _ROW_CHUNKS:
            chunks.append((b, r0, nr))
    n = len(chunks)
    isems = (isem0, isem1)
    osems = (osem0, osem1)

    def in_cp(i):
        b, r0, nr = chunks[i]
        return pltpu.make_async_copy(
            x_hbm.at[b0 + b, pl.ds(r0, nr), :],
            shared.at[sid, i % 2, pl.ds(0, nr), :],
            isems[i % 2])

    def out_cp(i):
        b, r0, nr = chunks[i]
        return pltpu.make_async_copy(
            shared.at[sid, i % 2, pl.ds(0, nr), :],
            o_hbm.at[b0 + b, pl.ds(r0, nr), :],
            osems[i % 2])

    def bnd_in(b):
        return pltpu.make_async_copy(
            x_hbm.at[b0 + b, pl.ds(_BND0, _BNDR), :], bbuf, bsem)

    def bnd_out(b):
        return pltpu.make_async_copy(
            bbuf, o_hbm.at[b0 + b, pl.ds(_BND0, _BNDR), :], bsem)

    def zero_band():
        def zr(i, _):
            r = _START - _BND0 + i // (_T // 16)
            c = (i % (_T // 16)) * 16
            bbuf[r, pl.ds(c, 16)] = jnp.zeros((16,), jnp.float32)
            return 0
        lax.fori_loop(0, _MASK * (_T // 16), zr, 0)

    bnd_in(0).start()
    in_cp(0).start()
    in_cp(1).start()

    for i in range(n):
        b, _, _ = chunks[i]
        pos = i % _NCH
        in_cp(i).wait()
        out_cp(i).start()
        if i + 2 < n:
            out_cp(i).wait()
            in_cp(i + 2).start()
        if pos == 3:
            bnd_in(b).wait()
            zero_band()
            bnd_out(b).start()
        elif pos == _NCH - 1 and b + 1 < _BPW:
            bnd_out(b).wait()
            bnd_in(b + 1).start()
    out_cp(n - 2).wait()
    out_cp(n - 1).wait()
    bnd_out(_BPW - 1).wait()


def _sc_mask_copy(x):
    k = functools.partial(
        pl.kernel,
        mesh=plsc.VectorSubcoreMesh(core_axis_name="c", subcore_axis_name="s"),
        out_type=jax.ShapeDtypeStruct((_B, _F, _T), jnp.float32),
        scratch_types=[
            pltpu.VMEM_SHARED((_NS, 2, _CR, _T), jnp.float32),
            pltpu.VMEM((_BNDR, _T), jnp.float32),
            pltpu.SemaphoreType.DMA,
            pltpu.SemaphoreType.DMA,
            pltpu.SemaphoreType.DMA,
            pltpu.SemaphoreType.DMA,
            pltpu.SemaphoreType.DMA,
        ],
    )(_sc_body)
    return k(x)


def kernel(x):
    return _sc_mask_copy(x)


# speedup vs baseline: 3.1797x; 1.0774x over previous
import functools

import jax
import jax.numpy as jnp
import numpy as np
from jax import lax
from jax.experimental import pallas as pl
from jax.experimental.pallas import tpu as pltpu
from jax.experimental.pallas import tpu_sc as plsc

_MAX_MASK_SIZE = 27
_rng = np.random.RandomState(0)
_MASK = int(_rng.randint(0, _MAX_MASK_SIZE))
_START = int(_rng.randint(0, 128 - _MASK))
_END = _START + _MASK

_B, _F, _T = 128, 128, 2048

_NC, _NS = 2, 16
_NW = _NC * _NS
_BPW = _B // _NW

_CR = 16
_ROW_CHUNKS = [
    (0, 16), (16, 16), (32, 8),
    (64, 16), (80, 16), (96, 16), (112, 16),
]
_NCH = len(_ROW_CHUNKS)
_G5, _G6, _G7 = 40, 48, 56
_BCHUNKS = [(_G5, range(_START - _G5, 8)), (_G7, range(0, _END - _G7))]


def _sc_body(x_hbm, o_hbm, shared, bbuf0, bbuf1, zbuf,
             isem0, isem1, osem0, osem1, bisem0, bisem1, bosem0, bosem1,
             zsem):
    cid = lax.axis_index("c")
    sid = lax.axis_index("s")
    wid = sid * _NC + cid
    b0 = wid * _BPW

    chunks = []
    for b in range(_BPW):
        for r0, nr in _ROW_CHUNKS:
            chunks.append((b, r0, nr))
    n = len(chunks)
    isems = (isem0, isem1)
    osems = (osem0, osem1)

    def in_cp(i):
        b, r0, nr = chunks[i]
        return pltpu.make_async_copy(
            x_hbm.at[b0 + b, pl.ds(r0, nr), :],
            shared.at[sid, i % 2, pl.ds(0, nr), :],
            isems[i % 2])

    def out_cp(i):
        b, r0, nr = chunks[i]
        return pltpu.make_async_copy(
            shared.at[sid, i % 2, pl.ds(0, nr), :],
            o_hbm.at[b0 + b, pl.ds(r0, nr), :],
            osems[i % 2])

    bchunks = [(b, g0, zrows) for b in range(_BPW) for g0, zrows in _BCHUNKS]
    nb = len(bchunks)
    bbufs = (bbuf0, bbuf1)
    bisems = (bisem0, bisem1)
    bosems = (bosem0, bosem1)

    def bnd_in(j):
        b, g0, _ = bchunks[j]
        return pltpu.make_async_copy(
            x_hbm.at[b0 + b, pl.ds(g0, 8), :], bbufs[j % 2], bisems[j % 2])

    def bnd_out(j):
        b, g0, _ = bchunks[j]
        return pltpu.make_async_copy(
            bbufs[j % 2], o_hbm.at[b0 + b, pl.ds(g0, 8), :], bosems[j % 2])

    def g6_out(b):
        return pltpu.make_async_copy(
            zbuf, o_hbm.at[b0 + b, pl.ds(_G6, 8), :], zsem)

    def zero_rows(buf, rows):
        def zr(i, _):
            r = rows.start + i // (_T // 16)
            c = (i % (_T // 16)) * 16
            buf[r, pl.ds(c, 16)] = jnp.zeros((16,), jnp.float32)
            return 0
        lax.fori_loop(0, len(rows) * (_T // 16), zr, 0)

    def band_step(j):
        if j >= 1:
            bnd_out(j - 1).wait()
            if j + 1 < nb:
                bnd_in(j + 1).start()
        b, _, zrows = bchunks[j]
        bnd_in(j).wait()
        zero_rows(bbufs[j % 2], zrows)
        bnd_out(j).start()
        if j % 2 == 0:
            g6_out(b).start()

    bnd_in(0).start()
    bnd_in(1).start()
    in_cp(0).start()
    in_cp(1).start()
    zero_rows(zbuf, range(0, 8))

    bj = 0
    for i in range(n):
        in_cp(i).wait()
        out_cp(i).start()
        if i + 2 < n:
            out_cp(i).wait()
            in_cp(i + 2).start()
        if i % 4 == 3 and bj < nb:
            band_step(bj)
            bj += 1
    while bj < nb:
        band_step(bj)
        bj += 1
    out_cp(n - 2).wait()
    out_cp(n - 1).wait()
    bnd_out(nb - 1).wait()
    for b in range(_BPW):
        g6_out(b).wait()


def _sc_mask_copy(x):
    k = functools.partial(
        pl.kernel,
        mesh=plsc.VectorSubcoreMesh(core_axis_name="c", subcore_axis_name="s"),
        out_type=jax.ShapeDtypeStruct((_B, _F, _T), jnp.float32),
        scratch_types=[
            pltpu.VMEM_SHARED((_NS, 2, _CR, _T), jnp.float32),
            pltpu.VMEM((8, _T), jnp.float32),
            pltpu.VMEM((8, _T), jnp.float32),
            pltpu.VMEM((8, _T), jnp.float32),
        ] + [pltpu.SemaphoreType.DMA] * 9,
    )(_sc_body)
    return k(x)


def kernel(x):
    return _sc_mask_copy(x)
